# Initial kernel scaffold; baseline (speedup 1.0000x reference)
import jax
import jax.numpy as jnp
from jax.experimental import pallas as pl


def kernel(embedding, edges, W_self, W_neigh, b_sage, W_cls, b_cls):
    raise NotImplementedError("write your pallas kernel here")


# speedup vs baseline: 39.1904x; 39.1904x over previous
import functools

import jax
import jax.numpy as jnp
from jax import lax
from jax.experimental import pallas as pl
from jax.experimental.pallas import tpu as pltpu
from jax.experimental.pallas import tpu_sc as plsc

N_NODES = 10000
N_EDGES = 320000
D = 128
NC = 2
NS = 16
NW = NC * NS
EPW = N_EDGES // NW
LANES = 16
CHUNKS = EPW // LANES


def _sc_filter_gather(emb_hbm, src_hbm, dst_hbm, sum_out, deg_out,
                      srcb, dstb, idxb, rowsb, accb, degb, degfb, sem):
    wid = lax.axis_index("s") * NC + lax.axis_index("c")
    base = wid * EPW
    pltpu.sync_copy(src_hbm.at[pl.ds(base, EPW)], srcb)
    pltpu.sync_copy(dst_hbm.at[pl.ds(base, EPW)], dstb)

    zf = jnp.zeros((LANES,), jnp.float32)
    for k in range(D // LANES):
        accb[pl.ds(k * LANES, LANES)] = zf
    degb[...] = jnp.zeros((LANES,), jnp.int32)

    def chunk(c, carry):
        off = c * LANES
        dv = dstb[pl.ds(off, LANES)]
        m = dv == 0
        cnt = jnp.sum(jnp.where(m, 1, 0).astype(jnp.int32))

        @pl.when(cnt > 0)
        def _():
            sv = srcb[pl.ds(off, LANES)]
            idxb[...] = jnp.where(m, sv, 0)
            wv = jnp.where(m, 1.0, 0.0).astype(jnp.float32)
            degb[...] = degb[...] + jnp.where(m, 1, 0).astype(jnp.int32)
            pltpu.async_copy(emb_hbm.at[idxb], rowsb, sem).wait()
            for r in range(LANES):
                wr = wv[r]
                for k in range(D // LANES):
                    sl = pl.ds(k * LANES, LANES)
                    accb[sl] = accb[sl] + wr * rowsb[r, sl]

        return carry

    lax.fori_loop(0, CHUNKS, chunk, 0)

    pltpu.sync_copy(accb, sum_out.at[wid])
    dt = jnp.sum(degb[...]).astype(jnp.float32)
    degfb[...] = jnp.full((LANES,), dt, jnp.float32)
    pltpu.sync_copy(degfb, deg_out.at[wid])


def _tc_finish(part_ref, deg_ref, emb_ref, ws_ref, wn_ref, bs_ref,
               wc_ref, bc_ref, out_ref):
    s = jnp.sum(part_ref[...], axis=0, keepdims=True)
    deg = jnp.sum(deg_ref[...], axis=0, keepdims=True)[0:1, 0:1]
    agg = s / jnp.maximum(deg, 1.0)
    e0 = emb_ref[0:1, :]
    h = jnp.maximum(
        jnp.dot(e0, ws_ref[...], preferred_element_type=jnp.float32)
        + jnp.dot(agg, wn_ref[...], preferred_element_type=jnp.float32)
        + bs_ref[...], 0.0)
    lg = jnp.dot(h, wc_ref[...], preferred_element_type=jnp.float32) + bc_ref[...]
    out_ref[...] = jnp.broadcast_to(lg, (8, 128))


def kernel(embedding, edges, W_self, W_neigh, b_sage, W_cls, b_cls):
    src = edges[0].astype(jnp.int32)
    dst = edges[1].astype(jnp.int32)

    mesh = plsc.VectorSubcoreMesh(core_axis_name="c", subcore_axis_name="s")
    sc_call = functools.partial(
        pl.kernel,
        mesh=mesh,
        compiler_params=pltpu.CompilerParams(needs_layout_passes=False),
        out_type=(
            jax.ShapeDtypeStruct((NW, D), jnp.float32),
            jax.ShapeDtypeStruct((NW, LANES), jnp.float32),
        ),
        scratch_types=[
            pltpu.VMEM((EPW,), jnp.int32),
            pltpu.VMEM((EPW,), jnp.int32),
            pltpu.VMEM((LANES,), jnp.int32),
            pltpu.VMEM((LANES, D), jnp.float32),
            pltpu.VMEM((D,), jnp.float32),
            pltpu.VMEM((LANES,), jnp.int32),
            pltpu.VMEM((LANES,), jnp.float32),
            pltpu.SemaphoreType.DMA,
        ],
    )
    partials, degs = sc_call(_sc_filter_gather)(embedding, src, dst)

    b_sage2 = b_sage.reshape(1, D)
    wc_pad = jnp.pad(W_cls, ((0, 0), (0, D - W_cls.shape[1])))
    bc_pad = jnp.pad(b_cls.reshape(1, -1), ((0, 0), (0, D - b_cls.shape[0])))

    out = pl.pallas_call(
        _tc_finish,
        out_shape=jax.ShapeDtypeStruct((8, D), jnp.float32),
        grid=(1,),
        in_specs=[
            pl.BlockSpec((NW, D), lambda i: (0, 0)),
            pl.BlockSpec((NW, LANES), lambda i: (0, 0)),
            pl.BlockSpec((8, D), lambda i: (0, 0)),
            pl.BlockSpec((D, D), lambda i: (0, 0)),
            pl.BlockSpec((D, D), lambda i: (0, 0)),
            pl.BlockSpec((1, D), lambda i: (0, 0)),
            pl.BlockSpec((D, D), lambda i: (0, 0)),
            pl.BlockSpec((1, D), lambda i: (0, 0)),
        ],
        out_specs=pl.BlockSpec((8, D), lambda i: (0, 0)),
    )(partials, degs, embedding, W_self, W_neigh, b_sage2, wc_pad, bc_pad)

    return out[0:1, 0:64]


# speedup vs baseline: 58.7520x; 1.4991x over previous
import functools

import jax
import jax.numpy as jnp
from jax import lax
from jax.experimental import pallas as pl
from jax.experimental.pallas import tpu as pltpu
from jax.experimental.pallas import tpu_sc as plsc

N_NODES = 10000
N_EDGES = 320000
D = 128
OUT = 64
NC = 2
NS = 16
NW = NC * NS
EPW = N_EDGES // NW
LANES = 16
CHUNKS = EPW // LANES
SCREEN = 25
NSCREEN = CHUNKS // SCREEN
WIN = ((EPW // 128) + 1) * 128


def _sc_filter_gather(emb_hbm, edges_hbm, sum_out, deg_out,
                      edgeb, idxb, rowsb, accb, degb, degfb, sem_e, sem_g):
    wid = lax.axis_index("s") * NC + lax.axis_index("c")
    base = wid * EPW
    ab = (base // 128) * 128
    delta = base - ab
    pltpu.async_copy(edges_hbm.at[:, pl.ds(ab, WIN)], edgeb, sem_e).wait()

    zf = jnp.zeros((LANES,), jnp.float32)
    for k in range(D // LANES):
        accb[pl.ds(k * LANES, LANES)] = zf
    degb[...] = jnp.zeros((LANES,), jnp.int32)

    def fine(c, carry):
        off = delta + c * LANES
        dv = edgeb[1, pl.ds(off, LANES)]
        m = dv == 0
        mi = jnp.where(m, 1, 0).astype(jnp.int32)
        cnt = jnp.sum(mi)

        @pl.when(cnt > 0)
        def _():
            sv = edgeb[0, pl.ds(off, LANES)]
            idxb[...] = jnp.zeros((LANES,), jnp.int32)
            plsc.store_compressed(idxb.at[pl.ds(0, LANES)], sv, mask=m)
            degb[...] = degb[...] + mi
            pltpu.async_copy(emb_hbm.at[idxb], rowsb, sem_g).wait()

            def acc_row(j, c2):
                for k in range(D // LANES):
                    sl = pl.ds(k * LANES, LANES)
                    accb[sl] = accb[sl] + rowsb[j, sl]
                return c2

            lax.fori_loop(0, cnt, acc_row, 0)

        return carry

    def screen(b, carry):
        off0 = delta + b * SCREEN * LANES
        mn = edgeb[1, pl.ds(off0, LANES)]
        for t in range(1, SCREEN):
            mn = jnp.minimum(mn, edgeb[1, pl.ds(off0 + t * LANES, LANES)])

        @pl.when(jnp.min(mn) == 0)
        def _():
            lax.fori_loop(b * SCREEN, (b + 1) * SCREEN, fine, 0)

        return carry

    lax.fori_loop(0, NSCREEN, screen, 0)

    pltpu.sync_copy(accb, sum_out.at[wid])
    dt = jnp.sum(degb[...]).astype(jnp.float32)
    degfb[...] = jnp.full((LANES,), dt, jnp.float32)
    pltpu.sync_copy(degfb, deg_out.at[wid])


def _tc_finish(part_ref, deg_ref, emb_ref, ws_ref, wn_ref, bs_ref,
               wc_ref, bc_ref, out_ref):
    s = jnp.sum(part_ref[...], axis=0, keepdims=True)
    deg = jnp.sum(deg_ref[...], axis=0, keepdims=True)[0:1, 0:1]
    agg = s / jnp.maximum(deg, 1.0)
    e0 = emb_ref[0:1, :]
    h = jnp.maximum(
        jnp.dot(e0, ws_ref[...], preferred_element_type=jnp.float32)
        + jnp.dot(agg, wn_ref[...], preferred_element_type=jnp.float32)
        + bs_ref[...][None, :], 0.0)
    out_ref[...] = (jnp.dot(h, wc_ref[...], preferred_element_type=jnp.float32)
                    + bc_ref[...][None, :])


def kernel(embedding, edges, W_self, W_neigh, b_sage, W_cls, b_cls):
    edges = edges.astype(jnp.int32)

    mesh = plsc.VectorSubcoreMesh(core_axis_name="c", subcore_axis_name="s")
    sc_call = functools.partial(
        pl.kernel,
        mesh=mesh,
        compiler_params=pltpu.CompilerParams(needs_layout_passes=False),
        out_type=(
            jax.ShapeDtypeStruct((NW, D), jnp.float32),
            jax.ShapeDtypeStruct((NW, LANES), jnp.float32),
        ),
        scratch_types=[
            pltpu.VMEM((2, WIN), jnp.int32),
            pltpu.VMEM((LANES,), jnp.int32),
            pltpu.VMEM((LANES, D), jnp.float32),
            pltpu.VMEM((D,), jnp.float32),
            pltpu.VMEM((LANES,), jnp.int32),
            pltpu.VMEM((LANES,), jnp.float32),
            pltpu.SemaphoreType.DMA,
            pltpu.SemaphoreType.DMA,
        ],
    )
    partials, degs = sc_call(_sc_filter_gather)(embedding, edges)

    out = pl.pallas_call(
        _tc_finish,
        out_shape=jax.ShapeDtypeStruct((1, OUT), jnp.float32),
        grid=(1,),
        in_specs=[
            pl.BlockSpec((NW, D), lambda i: (0, 0)),
            pl.BlockSpec((NW, LANES), lambda i: (0, 0)),
            pl.BlockSpec((8, D), lambda i: (0, 0)),
            pl.BlockSpec((D, D), lambda i: (0, 0)),
            pl.BlockSpec((D, D), lambda i: (0, 0)),
            pl.BlockSpec((D,), lambda i: (0,)),
            pl.BlockSpec((D, OUT), lambda i: (0, 0)),
            pl.BlockSpec((OUT,), lambda i: (0,)),
        ],
        out_specs=pl.BlockSpec((1, OUT), lambda i: (0, 0)),
    )(partials, degs, embedding, W_self, W_neigh, b_sage, W_cls, b_cls)

    return out
